# baseline (speedup 1.0000x reference)
import functools

import jax
import jax.numpy as jnp
from jax import lax
from jax.experimental import pallas as pl
from jax.experimental.pallas import tpu as pltpu
from jax.experimental.pallas import tpu_sc as plsc

_B, _S, _D = 4, 8192, 768
_NW = 32
_SLAB = _S // _NW
_CH = 32
_NRING = 4
_NCH = _SLAB // _CH


def _sc_body(in_hbm, pos_hbm, out_hbm,
             in0, in1, in2, in3, p0,
             r0, r1, r2, r3, w0, w1, w2, w3, ps0):
    wid = lax.axis_index("s") * 2 + lax.axis_index("c")
    s_base = wid * _SLAB
    n_vec = _D // 16

    in_bufs = (in0, in1, in2, in3)
    rsem = (r0, r1, r2, r3)
    wsem = (w0, w1, w2, w3)

    units = [(c, b) for c in range(_NCH) for b in range(_B)]
    nu = len(units)
    read_h = [None] * nu
    write_h = [None] * nu

    def issue_read(u):
        c, b = units[u]
        off = s_base + c * _CH
        read_h[u] = pltpu.async_copy(
            in_hbm.at[b, pl.ds(off, _CH)], in_bufs[u % _NRING], rsem[u % _NRING])

    for u in range(min(3, nu)):
        issue_read(u)

    for u in range(nu):
        c, b = units[u]
        if b == 0:
            pltpu.sync_copy(pos_hbm.at[pl.ds(s_base + c * _CH, _CH)], p0)
        read_h[u].wait()

        ib = in_bufs[u % _NRING]

        def _row(i, _):
            for k in range(n_vec):
                sl = pl.ds(k * 16, 16)
                ib[i, sl] = ib[i, sl] + p0[i, sl]
            return _
        lax.fori_loop(0, _CH, _row, None)

        off = s_base + c * _CH
        write_h[u] = pltpu.async_copy(
            ib, out_hbm.at[b, pl.ds(off, _CH)], wsem[u % _NRING])

        if u + 3 < nu:
            if u >= 1:
                write_h[u - 1].wait()
            issue_read(u + 3)

    for u in range(max(0, nu - 4), nu):
        write_h[u].wait()


def kernel(inputs, pos_table):
    mesh = plsc.VectorSubcoreMesh(core_axis_name="c", subcore_axis_name="s")
    f = functools.partial(
        pl.kernel,
        mesh=mesh,
        out_type=jax.ShapeDtypeStruct((_B, _S, _D), jnp.float32),
        scratch_types=(
            [pltpu.VMEM((_CH, _D), jnp.float32)] * 5
            + [pltpu.SemaphoreType.DMA] * 9
        ),
    )(_sc_body)
    return f(inputs, pos_table)


# speedup vs baseline: 1.7468x; 1.7468x over previous
import jax
import jax.numpy as jnp
from jax.experimental import pallas as pl


def _add_kernel(x_ref, p_ref, o_ref):
    o_ref[...] = x_ref[...] + p_ref[...]


def kernel(inputs, pos_table):
    B, S, D = inputs.shape
    BS = 1024

    grid = (S // BS,)
    return pl.pallas_call(
        _add_kernel,
        grid=grid,
        in_specs=[
            pl.BlockSpec((B, BS, D), lambda s: (0, s, 0)),
            pl.BlockSpec((BS, D), lambda s: (s, 0)),
        ],
        out_specs=pl.BlockSpec((B, BS, D), lambda s: (0, s, 0)),
        out_shape=jax.ShapeDtypeStruct((B, S, D), inputs.dtype),
    )(inputs, pos_table)
